# baseline (speedup 1.0000x reference)
import jax
import jax.numpy as jnp
from jax import lax
from jax.experimental import pallas as pl
from jax.experimental.pallas import tpu as pltpu
from jax.experimental.pallas import tpu_sc as plsc

M, N, D = 8192, 4, 2048
ND = N * D
NC, NS = 2, 16
NW = NC * NS
RW = M // NW
T = 8
NCHUNK = RW // T
VEC = 32


def _sc_body(res_hbm, h_hbm, out_hbm, h_v,
             res_v0, res_v1, out_v0, out_v1,
             sem_h, sem_in0, sem_in1, sem_out0, sem_out1):
    res_bufs = (res_v0, res_v1)
    out_bufs = (out_v0, out_v1)
    sems_in = (sem_in0, sem_in1)
    sems_out = (sem_out0, sem_out1)
    wid = lax.axis_index("s") * NC + lax.axis_index("c")
    row0 = wid * RW

    pltpu.async_copy(h_hbm.at[pl.ds(row0 * N, RW * N)], h_v, sem_h).wait()

    def load(k, buf):
        return pltpu.async_copy(
            res_hbm.at[pl.ds(row0 + k * T, T)], res_bufs[buf], sems_in[buf])

    def store(k, buf):
        return pltpu.async_copy(
            out_bufs[buf], out_hbm.at[pl.ds(row0 + k * T, T)], sems_out[buf])

    def wait_load(buf):
        pltpu.make_async_copy(
            res_hbm.at[pl.ds(row0, T)], res_bufs[buf], sems_in[buf]).wait()

    def wait_store(buf):
        pltpu.make_async_copy(
            out_bufs[buf], out_hbm.at[pl.ds(row0, T)], sems_out[buf]).wait()

    def compute(k, buf):
        rv = res_bufs[buf]
        ov = out_bufs[buf]
        hv = [h_v[pl.ds(k * T * N + 16 * i, 16)] for i in range(T * N // 16)]
        for t in range(T):
            hvec = hv[(t * N) // 16]
            w = [hvec[(t * N) % 16 + n] for n in range(N)]

            @pl.loop(0, D // VEC, unroll=8)
            def _(g):
                fmt = plsc.PackFormat.INTERLEAVED
                un = [plsc.unpack(rv[t, n * (D // VEC) + g], format=fmt)
                      for n in range(N)]
                acc_a = (un[0][0] * w[0] + un[1][0] * w[1]) + (un[2][0] * w[2] + un[3][0] * w[3])
                acc_b = (un[0][1] * w[0] + un[1][1] * w[1]) + (un[2][1] * w[2] + un[3][1] * w[3])
                ov[t, g] = plsc.pack(acc_a, acc_b, format=fmt)

    load(0, 0)

    @pl.loop(0, NCHUNK, step=2)
    def _(k):
        for b in range(2):
            kk = k + b

            @pl.when(kk + 1 < NCHUNK)
            def _():
                load(kk + 1, 1 - b)

            wait_load(b)

            @pl.when(kk >= 2)
            def _():
                wait_store(b)

            compute(kk, b)
            store(kk, b)

    for b in range(2):
        wait_store(b)


def kernel(res, h_pre):
    res2 = res.reshape(M, ND // VEC, VEC)
    h2 = h_pre.reshape(M * N)
    mesh = plsc.VectorSubcoreMesh(core_axis_name="c", subcore_axis_name="s")
    f = pl.kernel(
        _sc_body,
        out_type=jax.ShapeDtypeStruct((M, D // VEC, VEC), jnp.bfloat16),
        mesh=mesh,
        compiler_params=pltpu.CompilerParams(
            needs_layout_passes=False, use_tc_tiling_on_sc=False),
        scratch_types=[
            pltpu.VMEM((RW * N,), jnp.float32),
            pltpu.VMEM((T, ND // VEC, VEC), jnp.bfloat16),
            pltpu.VMEM((T, ND // VEC, VEC), jnp.bfloat16),
            pltpu.VMEM((T, D // VEC, VEC), jnp.bfloat16),
            pltpu.VMEM((T, D // VEC, VEC), jnp.bfloat16),
            pltpu.SemaphoreType.DMA,
            pltpu.SemaphoreType.DMA,
            pltpu.SemaphoreType.DMA,
            pltpu.SemaphoreType.DMA,
            pltpu.SemaphoreType.DMA,
        ],
    )
    return f(res2, h2).reshape(M, D)


# speedup vs baseline: 1.0531x; 1.0531x over previous
import jax
import jax.numpy as jnp
from jax import lax
from jax.experimental import pallas as pl
from jax.experimental.pallas import tpu as pltpu
from jax.experimental.pallas import tpu_sc as plsc

M, N, D = 8192, 4, 2048
ND = N * D
NC, NS = 2, 16
NW = NC * NS
RW = M // NW
T = 8
NCHUNK = RW // T
VEC = 32
GPR = D // VEC
GPN = ND // VEC


def _sc_body(res_hbm, h_hbm, out_hbm, h_v, wv_v,
             res_v0, res_v1, out_v0, out_v1,
             sem_h, sem_in0, sem_in1, sem_out0, sem_out1):
    res_bufs = (res_v0, res_v1)
    out_bufs = (out_v0, out_v1)
    sems_in = (sem_in0, sem_in1)
    sems_out = (sem_out0, sem_out1)
    wid = lax.axis_index("s") * NC + lax.axis_index("c")
    row0 = wid * RW

    pltpu.async_copy(h_hbm.at[pl.ds(row0 * N, RW * N)], h_v, sem_h).wait()

    @pl.loop(0, RW * N // 16)
    def _(i):
        hvec = h_v[pl.ds(i * 16, 16)]
        for j in range(16):
            b = jnp.full((16,), 1.0, jnp.float32) * hvec[j]
            wv_v[i * 16 + j] = plsc.pack(b, b, format=plsc.PackFormat.INTERLEAVED)

    def load(k, buf):
        return pltpu.async_copy(
            res_hbm.at[pl.ds(row0 + k * T, T)], res_bufs[buf], sems_in[buf])

    def store(k, buf):
        return pltpu.async_copy(
            out_bufs[buf], out_hbm.at[pl.ds(row0 + k * T, T)], sems_out[buf])

    def wait_load(buf):
        pltpu.make_async_copy(
            res_hbm.at[pl.ds(row0, T)], res_bufs[buf], sems_in[buf]).wait()

    def wait_store(buf):
        pltpu.make_async_copy(
            out_bufs[buf], out_hbm.at[pl.ds(row0, T)], sems_out[buf]).wait()

    def compute(k, buf):
        rv = res_bufs[buf]
        ov = out_bufs[buf]
        for t in range(T):
            row = k * T + t
            w = [wv_v[row * N + n] for n in range(N)]

            @pl.loop(0, GPR, unroll=8)
            def _(g):
                r = [rv[t, n * GPR + g] for n in range(N)]
                acc = (r[0] * w[0] + r[1] * w[1]) + (r[2] * w[2] + r[3] * w[3])
                ov[t, g] = acc

    load(0, 0)

    @pl.loop(0, NCHUNK, step=2)
    def _(k):
        for b in range(2):
            kk = k + b

            @pl.when(kk + 1 < NCHUNK)
            def _():
                load(kk + 1, 1 - b)

            wait_load(b)

            @pl.when(kk >= 2)
            def _():
                wait_store(b)

            compute(kk, b)
            store(kk, b)

    for b in range(2):
        wait_store(b)


def kernel(res, h_pre):
    res2 = res.reshape(M, GPN, VEC)
    h2 = h_pre.reshape(M * N)
    mesh = plsc.VectorSubcoreMesh(core_axis_name="c", subcore_axis_name="s")
    f = pl.kernel(
        _sc_body,
        out_type=jax.ShapeDtypeStruct((M, GPR, VEC), jnp.bfloat16),
        mesh=mesh,
        compiler_params=pltpu.CompilerParams(
            needs_layout_passes=False, use_tc_tiling_on_sc=False),
        scratch_types=[
            pltpu.VMEM((RW * N,), jnp.float32),
            pltpu.VMEM((RW * N, VEC), jnp.bfloat16),
            pltpu.VMEM((T, GPN, VEC), jnp.bfloat16),
            pltpu.VMEM((T, GPN, VEC), jnp.bfloat16),
            pltpu.VMEM((T, GPR, VEC), jnp.bfloat16),
            pltpu.VMEM((T, GPR, VEC), jnp.bfloat16),
            pltpu.SemaphoreType.DMA,
            pltpu.SemaphoreType.DMA,
            pltpu.SemaphoreType.DMA,
            pltpu.SemaphoreType.DMA,
            pltpu.SemaphoreType.DMA,
        ],
    )
    return f(res2, h2).reshape(M, D)


# speedup vs baseline: 1.4694x; 1.3953x over previous
import jax
import jax.numpy as jnp
from jax import lax
from jax.experimental import pallas as pl
from jax.experimental.pallas import tpu as pltpu
from jax.experimental.pallas import tpu_sc as plsc

M, N, D = 8192, 4, 2048
NC, NS = 2, 16
NW = NC * NS
RW = M // NW
T = 8
NCHUNK = RW // T
CT = D // 128
WPR = N * D * 2 // 512
OPR = 8 * D * 2 // 512
FMT = plsc.PackFormat.INTERLEAVED


def _sc_body(res_hbm, wq_hbm, out_hbm, wq_v,
             res_v0, res_v1, out_v0, out_v1,
             sem_w, sem_in0, sem_in1, sem_out0, sem_out1):
    res_bufs = (res_v0, res_v1)
    out_bufs = (out_v0, out_v1)
    sems_in = (sem_in0, sem_in1)
    sems_out = (sem_out0, sem_out1)
    wid = lax.axis_index("s") * NC + lax.axis_index("c")
    row0 = wid * RW

    pltpu.async_copy(wq_hbm.at[pl.ds(row0 // 64, RW // 64)], wq_v, sem_w).wait()

    def load(k, buf):
        return pltpu.async_copy(
            res_hbm.at[pl.ds((row0 + k * T) * WPR, T * WPR)],
            res_bufs[buf], sems_in[buf])

    def store(k, buf):
        return pltpu.async_copy(
            out_bufs[buf],
            out_hbm.at[pl.ds((row0 + k * T) * OPR // 8, OPR)], sems_out[buf])

    def wait_load(buf):
        pltpu.make_async_copy(
            res_hbm.at[pl.ds(row0 * WPR, T * WPR)], res_bufs[buf],
            sems_in[buf]).wait()

    def wait_store(buf):
        pltpu.make_async_copy(
            out_bufs[buf], out_hbm.at[pl.ds(row0 * OPR // 8, OPR)],
            sems_out[buf]).wait()

    def compute(k, buf):
        rv = res_bufs[buf]
        ov = out_bufs[buf]
        wrow = (k * T) // 64
        wcol = 2 * ((k * T) % 64)
        wv16 = wq_v[wrow, pl.ds(wcol, 16)]
        wA = [plsc.bitcast(jnp.full((16,), 1, jnp.int32) * wv16[2 * t],
                           jnp.bfloat16) for t in range(T)]
        wB = [plsc.bitcast(jnp.full((16,), 1, jnp.int32) * wv16[2 * t + 1],
                           jnp.bfloat16) for t in range(T)]

        @pl.loop(0, CT)
        def _(c):
            for u in range(T // 2):
                for j in range(8):
                    o = [None, None]
                    for p in range(2):
                        t = 2 * u + p
                        a = plsc.bitcast(
                            rv[t * WPR + c * 2, pl.ds(j * 16, 16)], jnp.bfloat16)
                        b = plsc.bitcast(
                            rv[t * WPR + c * 2 + 1, pl.ds(j * 16, 16)], jnp.bfloat16)
                        v = a * wA[t] + b * wB[t]
                        x, y = plsc.unpack(v, format=FMT)
                        o[p] = x + y
                    ov[c * 4 + u, pl.ds(j * 16, 16)] = plsc.bitcast(
                        plsc.pack(o[0], o[1], format=FMT), jnp.int32)

    load(0, 0)

    @pl.loop(0, NCHUNK, step=2)
    def _(k):
        for b in range(2):
            kk = k + b

            @pl.when(kk + 1 < NCHUNK)
            def _():
                load(kk + 1, 1 - b)

            wait_load(b)

            @pl.when(kk >= 2)
            def _():
                wait_store(b)

            compute(kk, b)
            store(kk, b)

    for b in range(2):
        wait_store(b)


def kernel(res, h_pre):
    rt = res.reshape(M, 2, 2, CT, 128).transpose(0, 3, 1, 4, 2)
    res_i = lax.bitcast_convert_type(rt, jnp.int32).reshape(M * WPR, 128)
    hb = h_pre.astype(jnp.bfloat16)
    wq = lax.bitcast_convert_type(hb.reshape(M, 2, 2), jnp.int32)
    wqr = wq.reshape(M // 64, 128)

    mesh = plsc.VectorSubcoreMesh(core_axis_name="c", subcore_axis_name="s")
    f = pl.kernel(
        _sc_body,
        out_type=jax.ShapeDtypeStruct((M * OPR // 8, 128), jnp.int32),
        mesh=mesh,
        compiler_params=pltpu.CompilerParams(
            needs_layout_passes=False, use_tc_tiling_on_sc=False),
        scratch_types=[
            pltpu.VMEM((RW // 64, 128), jnp.int32),
            pltpu.VMEM((T * WPR, 128), jnp.int32),
            pltpu.VMEM((T * WPR, 128), jnp.int32),
            pltpu.VMEM((OPR, 128), jnp.int32),
            pltpu.VMEM((OPR, 128), jnp.int32),
            pltpu.SemaphoreType.DMA,
            pltpu.SemaphoreType.DMA,
            pltpu.SemaphoreType.DMA,
            pltpu.SemaphoreType.DMA,
            pltpu.SemaphoreType.DMA,
        ],
    )
    oi = f(res_i, wqr)
    ob = lax.bitcast_convert_type(oi.reshape(M // 8, CT, 4, 128), jnp.bfloat16)
    return ob.transpose(0, 2, 4, 1, 3).reshape(M, D)


# speedup vs baseline: 2.4124x; 1.6418x over previous
import jax
import jax.numpy as jnp
from jax import lax
from jax.experimental import pallas as pl
from jax.experimental.pallas import tpu as pltpu
from jax.experimental.pallas import tpu_sc as plsc

M, N, D = 8192, 4, 2048
NC, NS = 2, 16
NW = NC * NS
RW = M // NW
T = 8
NCHUNK = RW // T
CT = D // 128
WPR = 2 * CT
OPR = CT * T // 2
FMT = plsc.PackFormat.INTERLEAVED


def _sc_body(res_hbm, wq_hbm, out_hbm, wq_v,
             res_v0, res_v1, out_v0, out_v1,
             sem_w, sem_in0, sem_in1, sem_out0, sem_out1):
    res_bufs = (res_v0, res_v1)
    out_bufs = (out_v0, out_v1)
    sems_in = (sem_in0, sem_in1)
    sems_out = (sem_out0, sem_out1)
    wid = lax.axis_index("s") * NC + lax.axis_index("c")
    row0 = wid * RW

    pltpu.async_copy(wq_hbm.at[pl.ds(row0 // 64, RW // 64)], wq_v, sem_w).wait()

    def load(k, buf):
        return pltpu.async_copy(
            res_hbm.at[pl.ds(row0 + k * T, T)],
            res_bufs[buf], sems_in[buf])

    def store(k, buf):
        return pltpu.async_copy(
            out_bufs[buf],
            out_hbm.at[pl.ds((row0 + k * T) * 8, OPR)], sems_out[buf])

    def wait_load(buf):
        pltpu.make_async_copy(
            res_hbm.at[pl.ds(row0, T)], res_bufs[buf],
            sems_in[buf]).wait()

    def wait_store(buf):
        pltpu.make_async_copy(
            out_bufs[buf], out_hbm.at[pl.ds(row0 * 8, OPR)],
            sems_out[buf]).wait()

    def compute(k, buf):
        rv = res_bufs[buf]
        ov = out_bufs[buf]
        wrow = (k * T) // 64
        wcol = 2 * ((k * T) % 64)
        wv16 = wq_v[wrow, pl.ds(wcol, 16)]
        wA = [plsc.bitcast(jnp.full((16,), 1, jnp.int32) * wv16[2 * t],
                           jnp.bfloat16) for t in range(T)]
        wB = [plsc.bitcast(jnp.full((16,), 1, jnp.int32) * wv16[2 * t + 1],
                           jnp.bfloat16) for t in range(T)]

        @pl.loop(0, CT)
        def _(c):
            for u in range(T // 2):
                for j in range(8):
                    o = [None, None]
                    for p in range(2):
                        t = 2 * u + p
                        a = plsc.bitcast(
                            rv[t, 0, c, pl.ds(j * 16, 16)], jnp.bfloat16)
                        b = plsc.bitcast(
                            rv[t, 1, c, pl.ds(j * 16, 16)], jnp.bfloat16)
                        v = a * wA[t] + b * wB[t]
                        x, y = plsc.unpack(v, format=FMT)
                        o[p] = x + y
                    ov[u * CT + c, pl.ds(j * 16, 16)] = plsc.bitcast(
                        plsc.pack(o[0], o[1], format=FMT), jnp.int32)

    load(0, 0)

    @pl.loop(0, NCHUNK, step=2)
    def _(k):
        for b in range(2):
            kk = k + b

            @pl.when(kk + 1 < NCHUNK)
            def _():
                load(kk + 1, 1 - b)

            wait_load(b)

            @pl.when(kk >= 2)
            def _():
                wait_store(b)

            compute(kk, b)
            store(kk, b)

    for b in range(2):
        wait_store(b)


def kernel(res, h_pre):
    xv = lax.bitcast_convert_type(res, jnp.uint16)
    lo = xv[:, 0::2, :].astype(jnp.uint32)
    hi = xv[:, 1::2, :].astype(jnp.uint32)
    w = lo | (hi << 16)
    res_i = lax.bitcast_convert_type(w, jnp.int32).reshape(M, 2, CT, 128)

    hb = h_pre.astype(jnp.bfloat16)
    wq = lax.bitcast_convert_type(hb.reshape(M, 2, 2), jnp.int32)
    wqr = wq.reshape(M // 64, 128)

    mesh = plsc.VectorSubcoreMesh(core_axis_name="c", subcore_axis_name="s")
    f = pl.kernel(
        _sc_body,
        out_type=jax.ShapeDtypeStruct((M * 8, 128), jnp.int32),
        mesh=mesh,
        compiler_params=pltpu.CompilerParams(
            needs_layout_passes=False, use_tc_tiling_on_sc=False),
        scratch_types=[
            pltpu.VMEM((RW // 64, 128), jnp.int32),
            pltpu.VMEM((T, 2, CT, 128), jnp.int32),
            pltpu.VMEM((T, 2, CT, 128), jnp.int32),
            pltpu.VMEM((OPR, 128), jnp.int32),
            pltpu.VMEM((OPR, 128), jnp.int32),
            pltpu.SemaphoreType.DMA,
            pltpu.SemaphoreType.DMA,
            pltpu.SemaphoreType.DMA,
            pltpu.SemaphoreType.DMA,
            pltpu.SemaphoreType.DMA,
        ],
    )
    oi = f(res_i, wqr)
    ou = lax.bitcast_convert_type(oi, jnp.uint32)
    olo = lax.bitcast_convert_type((ou & 0xFFFF).astype(jnp.uint16),
                                   jnp.bfloat16).reshape(M // 2, 1, D)
    ohi = lax.bitcast_convert_type((ou >> 16).astype(jnp.uint16),
                                   jnp.bfloat16).reshape(M // 2, 1, D)
    return jnp.concatenate([olo, ohi], axis=1).reshape(M, D)
